# baseline (speedup 1.0000x reference)
import functools

import jax
import jax.numpy as jnp
from jax import lax
from jax.experimental import pallas as pl
from jax.experimental.pallas import tpu as pltpu
from jax.experimental.pallas import tpu_sc as plsc

_B = 8
_H = 512
_W = 512
_KSC = 2
_NTILES = 32


def _tc_body(g_ref, p_ref, m_ref, s_ref, c_ref):
    i = pl.program_id(0)

    @pl.when(i == 0)
    def _init():
        s_ref[0, 0] = 0.0
        c_ref[0, 0] = 0.0

    dx = p_ref[0] - g_ref[0]
    dy = p_ref[1] - g_ref[1]
    dz = p_ref[2] - g_ref[2]
    d2 = dx * dx + dy * dy + dz * dz
    dist = jnp.sqrt(d2)
    mf = (m_ref[0] != 0).astype(jnp.float32)
    s_ref[0, 0] += jnp.sum(dist * mf)
    c_ref[0, 0] += jnp.sum(mf)


def _tc_partial(gp, pp, m8, nb):
    plane_spec = pl.BlockSpec((3, _H, _W), lambda i: (i, 0, 0))
    mask_spec = pl.BlockSpec((1, _H, _W), lambda i: (i, 0, 0))
    scalar_spec = pl.BlockSpec(memory_space=pltpu.SMEM)
    s, c = pl.pallas_call(
        _tc_body,
        grid=(nb,),
        in_specs=[plane_spec, plane_spec, mask_spec],
        out_specs=[scalar_spec, scalar_spec],
        out_shape=[
            jax.ShapeDtypeStruct((1, 1), jnp.float32),
            jax.ShapeDtypeStruct((1, 1), jnp.float32),
        ],
    )(gp, pp, m8)
    return s[0, 0], c[0, 0]


def _sc_partial(gp, pp, mf32, b0, nb):
    mesh = plsc.VectorSubcoreMesh(core_axis_name="c", subcore_axis_name="s")
    f32 = jnp.float32

    @functools.partial(
        pl.kernel,
        mesh=mesh,
        out_type=[
            jax.ShapeDtypeStruct((_NTILES, 16), f32),
            jax.ShapeDtypeStruct((_NTILES, 16), f32),
        ],
        scratch_types=[
            pltpu.VMEM((2, 7, 16, _W), f32),
            pltpu.VMEM((16,), f32), pltpu.VMEM((16,), f32),
            pltpu.SemaphoreType.DMA, pltpu.SemaphoreType.DMA,
        ],
    )
    def k(gp_h, pp_h, m_h, outs_h, outc_h, buf, vs, vc, sem0, sem1):
        cid = lax.axis_index("c")
        sid = lax.axis_index("s")
        w = sid * 2 + cid
        r0 = 16 * w
        sems = (sem0, sem1)

        def start_round(j, b):
            sem = sems[j % 2]
            dst = buf.at[j % 2]
            for c in range(3):
                pltpu.async_copy(gp_h.at[3 * b + c, pl.ds(r0, 16), :],
                                 dst.at[c], sem)
                pltpu.async_copy(pp_h.at[3 * b + c, pl.ds(r0, 16), :],
                                 dst.at[3 + c], sem)
            pltpu.async_copy(m_h.at[j, pl.ds(r0, 16), :], dst.at[6], sem)

        def drain_round(j, b):
            sem = sems[j % 2]
            dst = buf.at[j % 2]
            for c in range(3):
                pltpu.make_async_copy(gp_h.at[3 * b + c, pl.ds(r0, 16), :],
                                      dst.at[c], sem).wait()
                pltpu.make_async_copy(pp_h.at[3 * b + c, pl.ds(r0, 16), :],
                                      dst.at[3 + c], sem).wait()
            pltpu.make_async_copy(m_h.at[j, pl.ds(r0, 16), :],
                                  dst.at[6], sem).wait()

        def make_row(jmod):
            bb = buf.at[jmod]

            def row(gi, carry):
                acc = list(carry)
                lt = gi >> 3
                r8 = gi & 7
                lr = (lt << 1) + (r8 >> 2)
                cb = (r8 & 3) << 7
                for s8 in range(8):
                    cc = pl.multiple_of(cb + (s8 << 4), 16)
                    xg = bb[0, lr, pl.ds(cc, 16)]
                    yg = bb[1, lr, pl.ds(cc, 16)]
                    zg = bb[2, lr, pl.ds(cc, 16)]
                    xp = bb[3, lr, pl.ds(cc, 16)]
                    yp = bb[4, lr, pl.ds(cc, 16)]
                    zp = bb[5, lr, pl.ds(cc, 16)]
                    mf = bb[6, lr, pl.ds(cc, 16)]
                    dx = xp - xg
                    dy = yp - yg
                    dz = zp - zg
                    d2 = jnp.maximum(dx * dx + dy * dy + dz * dz, f32(1e-24))
                    ii = lax.bitcast_convert_type(d2, jnp.int32)
                    ri = jnp.int32(0x5F3759DF) - lax.shift_right_logical(ii, 1)
                    r = lax.bitcast_convert_type(ri, f32)
                    r = r * (f32(1.5) - f32(0.5) * d2 * r * r)
                    r = r * (f32(1.5) - f32(0.5) * d2 * r * r)
                    r = r * (f32(1.5) - f32(0.5) * d2 * r * r)
                    q = s8 & 3
                    acc[q] = acc[q] + (d2 * r) * mf
                    acc[4 + q] = acc[4 + q] + mf
                return tuple(acc)

            return row

        acc = tuple(jnp.zeros((16,), f32) for _ in range(8))
        start_round(0, b0)
        for j in range(nb):
            b = b0 + j
            drain_round(j, b)
            if j + 1 < nb:
                start_round(j + 1, b + 1)
            acc = lax.fori_loop(0, 64, make_row(j % 2), acc)
        vs[...] = acc[0] + acc[1] + acc[2] + acc[3]
        vc[...] = acc[4] + acc[5] + acc[6] + acc[7]
        pltpu.sync_copy(vs, outs_h.at[w])
        pltpu.sync_copy(vc, outc_h.at[w])

    outs, outc = k(gp, pp, mf32)
    return jnp.sum(outs), jnp.sum(outc)


def kernel(gt_pts3d, pred_pts3d, valid_mask):
    gp = jnp.transpose(gt_pts3d, (0, 3, 1, 2)).reshape(3 * _B, _H, _W)
    pp = jnp.transpose(pred_pts3d, (0, 3, 1, 2)).reshape(3 * _B, _H, _W)
    m8 = valid_mask.view(jnp.int8)
    mf32 = valid_mask[_B - _KSC:].astype(jnp.float32)

    s_tc, c_tc = _tc_partial(gp, pp, m8, _B - _KSC)
    s_sc, c_sc = _sc_partial(gp, pp, mf32, _B - _KSC, _KSC)
    tot = s_tc + s_sc
    cnt = c_tc + c_sc
    l = jnp.where(cnt > 0.0, tot / jnp.maximum(cnt, 1.0),
                  jnp.zeros((), jnp.float32))
    return (l, valid_mask)


# speedup vs baseline: 1.8630x; 1.8630x over previous
import jax
import jax.numpy as jnp
from jax.experimental import pallas as pl
from jax.experimental.pallas import tpu as pltpu

_B = 8
_H = 512
_W = 512


def _body(g_ref, p_ref, m_ref, s_ref, c_ref, l_ref):
    i = pl.program_id(0)

    @pl.when(i == 0)
    def _init():
        s_ref[0, 0] = 0.0
        c_ref[0, 0] = 0.0
        l_ref[0, 0] = 0.0

    dx = p_ref[0] - g_ref[0]
    dy = p_ref[1] - g_ref[1]
    dz = p_ref[2] - g_ref[2]
    d2 = dx * dx + dy * dy + dz * dz
    dist = jnp.sqrt(d2)
    mf = (m_ref[0] != 0).astype(jnp.float32)
    s_ref[0, 0] += jnp.sum(dist * mf)
    c_ref[0, 0] += jnp.sum(mf)

    @pl.when(i == pl.num_programs(0) - 1)
    def _fin():
        cnt = c_ref[0, 0]
        tot = s_ref[0, 0]
        l_ref[0, 0] = jnp.where(cnt > 0.0, tot / jnp.maximum(cnt, 1.0), 0.0)


def kernel(gt_pts3d, pred_pts3d, valid_mask):
    gp = jnp.transpose(gt_pts3d, (0, 3, 1, 2)).reshape(3 * _B, _H, _W)
    pp = jnp.transpose(pred_pts3d, (0, 3, 1, 2)).reshape(3 * _B, _H, _W)

    plane_spec = pl.BlockSpec((3, _H, _W), lambda i: (i, 0, 0))
    mask_spec = pl.BlockSpec((1, _H, _W), lambda i: (i, 0, 0))
    scalar_spec = pl.BlockSpec(memory_space=pltpu.SMEM)
    _, _, l = pl.pallas_call(
        _body,
        grid=(_B,),
        in_specs=[plane_spec, plane_spec, mask_spec],
        out_specs=[scalar_spec, scalar_spec, scalar_spec],
        out_shape=[
            jax.ShapeDtypeStruct((1, 1), jnp.float32),
            jax.ShapeDtypeStruct((1, 1), jnp.float32),
            jax.ShapeDtypeStruct((1, 1), jnp.float32),
        ],
    )(gp, pp, valid_mask.view(jnp.int8))
    return (l[0, 0], valid_mask)
